# baseline (speedup 1.0000x reference)
import functools

import jax
import jax.numpy as jnp
from jax import lax
from jax.experimental import pallas as pl
from jax.experimental.pallas import tpu as pltpu
from jax.experimental.pallas import tpu_sc as plsc

_L = 16


def _gate_body(ew_hbm, sc_hbm, ei_hbm, ab_hbm, out_hbm,
               scores_v, ei0, ei1, ei2, ew0, ew1, ew2, out0, out1, ab_v,
               sin0, sin1, sin2, sout0, sout1,
               *, chunk, num_cores, n_hi, n_lo_workers):
    wid = lax.axis_index("s") * num_cores + lax.axis_index("c")
    n_w = jnp.where(wid < n_lo_workers, n_hi, n_hi - 1)
    first = wid * (n_hi - 1) + jnp.minimum(wid, n_lo_workers)
    n_groups = (n_hi - 1) // 6
    tail = n_w - 6 * n_groups

    eis = (ei0, ei1, ei2)
    ews = (ew0, ew1, ew2)
    outs = (out0, out1)
    sins = (sin0, sin1, sin2)
    souts = (sout0, sout1)

    pltpu.sync_copy(sc_hbm, scores_v)
    pltpu.sync_copy(ab_hbm, ab_v)
    na = ab_v[0]
    nb = ab_v[1]

    def in_descs(b, c):
        base = (first + c) * chunk
        return (pltpu.make_async_copy(ei_hbm.at[:, pl.ds(base, chunk)],
                                      eis[b], sins[b]),
                pltpu.make_async_copy(ew_hbm.at[pl.ds(base, chunk)],
                                     ews[b], sins[b]))

    def start_in(b, c):
        d1, d2 = in_descs(b, c)
        d1.start()
        d2.start()

    def wait_in(b, c):
        d1, d2 = in_descs(b, c)
        d1.wait()
        d2.wait()

    def out_desc(b, c):
        base = (first + c) * chunk
        return pltpu.make_async_copy(outs[b], out_hbm.at[pl.ds(base, chunk)],
                                     souts[b])

    def compute(ei_v, ew_v, out_v):
        @plsc.parallel_loop(0, chunk // _L, unroll=8)
        def _(v):
            sl = pl.ds(v * _L, _L)
            si = plsc.load_gather(scores_v, [ei_v[0, sl]])
            sj = plsc.load_gather(scores_v, [ei_v[1, sl]])
            x = na * (si + sj) + nb
            out_v[sl] = ew_v[sl] / (1.0 + jnp.exp(x))

    start_in(0, 0)
    start_in(1, 1)
    start_in(2, 2)

    def group_body(g, carry):
        c0 = 6 * g
        for k in range(6):
            c = c0 + k
            bi = k % 3
            bo = k % 2
            wait_in(bi, c)
            if k >= 2:
                out_desc(bo, c - 2).wait()
            else:
                @pl.when(g > 0)
                def _():
                    out_desc(bo, c - 2).wait()
            compute(eis[bi], ews[bi], outs[bo])
            out_desc(bo, c).start()

            @pl.when(c + 3 < n_w)
            def _():
                start_in(bi, c + 3)

        return carry

    lax.fori_loop(0, n_groups, group_body, 0)

    @pl.when(tail > 0)
    def _():
        c = 6 * n_groups
        wait_in(0, c)
        out_desc(0, c - 2).wait()
        compute(ei0, ew0, out0)
        out_desc(0, c).start()

    last0 = jnp.where(tail > 0, 6 * n_groups, 6 * n_groups - 2)
    out_desc(0, last0).wait()
    out_desc(1, 6 * n_groups - 1).wait()


def kernel(edge_weights, scores, edge_index, alpha, beta):
    E = edge_weights.shape[0]
    info = plsc.get_sparse_core_info()
    nw = info.num_cores * info.num_subcores
    chunk = 1280
    assert E % chunk == 0 and chunk % 128 == 0
    total_chunks = E // chunk
    n_hi = -(-total_chunks // nw)
    n_lo_workers = total_chunks - nw * (n_hi - 1)
    assert (n_hi - 1) % 6 == 0 or n_lo_workers == nw

    ei = edge_index.astype(jnp.int32)
    ab = jnp.stack([jnp.full((_L,), -alpha, jnp.float32),
                    jnp.full((_L,), -beta, jnp.float32)])

    mesh = plsc.VectorSubcoreMesh(core_axis_name="c", subcore_axis_name="s")
    body = functools.partial(_gate_body, chunk=chunk,
                             num_cores=info.num_cores,
                             n_hi=n_hi, n_lo_workers=n_lo_workers)
    run = pl.kernel(
        body,
        mesh=mesh,
        compiler_params=pltpu.CompilerParams(needs_layout_passes=False),
        out_type=jax.ShapeDtypeStruct((E,), jnp.float32),
        scratch_types=[
            pltpu.VMEM((scores.shape[0],), jnp.float32),
            pltpu.VMEM((2, chunk), jnp.int32),
            pltpu.VMEM((2, chunk), jnp.int32),
            pltpu.VMEM((2, chunk), jnp.int32),
            pltpu.VMEM((chunk,), jnp.float32),
            pltpu.VMEM((chunk,), jnp.float32),
            pltpu.VMEM((chunk,), jnp.float32),
            pltpu.VMEM((chunk,), jnp.float32),
            pltpu.VMEM((chunk,), jnp.float32),
            pltpu.VMEM((2, _L), jnp.float32),
            pltpu.SemaphoreType.DMA,
            pltpu.SemaphoreType.DMA,
            pltpu.SemaphoreType.DMA,
            pltpu.SemaphoreType.DMA,
            pltpu.SemaphoreType.DMA,
        ],
    )
    return run(edge_weights, scores, ei, ab)


# speedup vs baseline: 1.2087x; 1.2087x over previous
import functools

import jax
import jax.numpy as jnp
from jax import lax
from jax.experimental import pallas as pl
from jax.experimental.pallas import tpu as pltpu
from jax.experimental.pallas import tpu_sc as plsc

_L = 16


def _gate_body(ew_hbm, sc_hbm, ei_hbm, ab_hbm, out_hbm,
               scores_v, ei0, ei1, ei2, ew0, ew1, ew2, out0, out1, ab_v,
               sin0, sin1, sin2, sout0, sout1,
               *, chunk, num_cores, n_hi, n_lo_workers):
    wid = lax.axis_index("s") * num_cores + lax.axis_index("c")
    n_w = jnp.where(wid < n_lo_workers, n_hi, n_hi - 1)
    first = wid * (n_hi - 1) + jnp.minimum(wid, n_lo_workers)
    n_groups = (n_hi - 1) // 6
    tail = n_w - 6 * n_groups

    eis = (ei0, ei1, ei2)
    ews = (ew0, ew1, ew2)
    outs = (out0, out1)
    sins = (sin0, sin1, sin2)
    souts = (sout0, sout1)

    pltpu.sync_copy(sc_hbm, scores_v)
    pltpu.sync_copy(ab_hbm, ab_v)
    na = ab_v[0]
    nb = ab_v[1]

    def in_descs(b, c):
        base = (first + c) * chunk
        return (pltpu.make_async_copy(ei_hbm.at[:, pl.ds(base, chunk)],
                                      eis[b], sins[b]),
                pltpu.make_async_copy(ew_hbm.at[pl.ds(base, chunk)],
                                     ews[b], sins[b]))

    def start_in(b, c):
        d1, d2 = in_descs(b, c)
        d1.start()
        d2.start()

    def wait_in(b, c):
        d1, d2 = in_descs(b, c)
        d1.wait()
        d2.wait()

    def out_desc(b, c):
        base = (first + c) * chunk
        return pltpu.make_async_copy(outs[b], out_hbm.at[pl.ds(base, chunk)],
                                     souts[b])

    def compute(ei_v, ew_v, out_v):
        @plsc.parallel_loop(0, chunk // _L, unroll=8)
        def _(v):
            sl = pl.ds(v * _L, _L)
            si = plsc.load_gather(scores_v, [ei_v[0, sl]])
            sj = plsc.load_gather(scores_v, [ei_v[1, sl]])
            x = na * (si + sj) + nb
            out_v[sl] = ew_v[sl] / (1.0 + jnp.exp(x))

    start_in(0, 0)
    start_in(1, 1)
    start_in(2, 2)

    def group_body(g, carry):
        c0 = 6 * g
        for k in range(6):
            c = c0 + k
            bi = k % 3
            bo = k % 2
            wait_in(bi, c)
            if k >= 2:
                out_desc(bo, c - 2).wait()
            else:
                @pl.when(g > 0)
                def _():
                    out_desc(bo, c - 2).wait()
            compute(eis[bi], ews[bi], outs[bo])
            out_desc(bo, c).start()

            @pl.when(c + 3 < n_w)
            def _():
                start_in(bi, c + 3)

        return carry

    lax.fori_loop(0, n_groups, group_body, 0)

    @pl.when(tail > 0)
    def _():
        c = 6 * n_groups
        wait_in(0, c)
        out_desc(0, c - 2).wait()
        compute(ei0, ew0, out0)
        out_desc(0, c).start()

    last0 = jnp.where(tail > 0, 6 * n_groups, 6 * n_groups - 2)
    out_desc(0, last0).wait()
    out_desc(1, 6 * n_groups - 1).wait()


def kernel(edge_weights, scores, edge_index, alpha, beta):
    E = edge_weights.shape[0]
    info = plsc.get_sparse_core_info()
    nw = info.num_cores * info.num_subcores
    chunk = 2560
    assert E % chunk == 0 and chunk % 128 == 0
    total_chunks = E // chunk
    n_hi = -(-total_chunks // nw)
    n_lo_workers = total_chunks - nw * (n_hi - 1)
    assert (n_hi - 1) % 6 == 0 or n_lo_workers == nw

    ei = edge_index.astype(jnp.int32)
    ab = jnp.stack([jnp.full((_L,), -alpha, jnp.float32),
                    jnp.full((_L,), -beta, jnp.float32)])

    mesh = plsc.VectorSubcoreMesh(core_axis_name="c", subcore_axis_name="s")
    body = functools.partial(_gate_body, chunk=chunk,
                             num_cores=info.num_cores,
                             n_hi=n_hi, n_lo_workers=n_lo_workers)
    run = pl.kernel(
        body,
        mesh=mesh,
        compiler_params=pltpu.CompilerParams(needs_layout_passes=False),
        out_type=jax.ShapeDtypeStruct((E,), jnp.float32),
        scratch_types=[
            pltpu.VMEM((scores.shape[0],), jnp.float32),
            pltpu.VMEM((2, chunk), jnp.int32),
            pltpu.VMEM((2, chunk), jnp.int32),
            pltpu.VMEM((2, chunk), jnp.int32),
            pltpu.VMEM((chunk,), jnp.float32),
            pltpu.VMEM((chunk,), jnp.float32),
            pltpu.VMEM((chunk,), jnp.float32),
            pltpu.VMEM((chunk,), jnp.float32),
            pltpu.VMEM((chunk,), jnp.float32),
            pltpu.VMEM((2, _L), jnp.float32),
            pltpu.SemaphoreType.DMA,
            pltpu.SemaphoreType.DMA,
            pltpu.SemaphoreType.DMA,
            pltpu.SemaphoreType.DMA,
            pltpu.SemaphoreType.DMA,
        ],
    )
    return run(edge_weights, scores, ei, ab)
